# baseline (speedup 1.0000x reference)
import functools

import jax
import jax.numpy as jnp
from jax import lax
from jax.experimental import pallas as pl
from jax.experimental.pallas import tpu as pltpu
from jax.experimental.pallas import tpu_sc as plsc

N_NODES = 100000
D = 64
L = 50
B = 4096
NR = 5

NTILES = 32
BPT = B // NTILES
NCHUNK = L
CH = BPT
PRE_BLK = 1000


def _pre_body(feat_ref, w2_ref, rtab_ref, wabot_ref, bagg_ref,
              p2_ref, c8_ref):
    p2_ref[...] = jnp.dot(feat_ref[...], w2_ref[...],
                          preferred_element_type=jnp.float32)
    ct = jnp.dot(rtab_ref[...], wabot_ref[...],
                 preferred_element_type=jnp.float32) + bagg_ref[...]
    ct = jnp.concatenate([ct, jnp.zeros((8 - NR, D), jnp.float32)], axis=0)
    c8_ref[...] = jnp.concatenate([ct, jnp.zeros((8, D), jnp.float32)],
                                  axis=1)


@jax.jit
def _tc_prelude(feat, w2, r_table, wa_bot, bagg2):
    return pl.pallas_call(
        _pre_body,
        grid=(N_NODES // PRE_BLK,),
        in_specs=[
            pl.BlockSpec((PRE_BLK, D), lambda j: (j, 0)),
            pl.BlockSpec((D, 2 * D), lambda j: (0, 0)),
            pl.BlockSpec((NR, D), lambda j: (0, 0)),
            pl.BlockSpec((D, D), lambda j: (0, 0)),
            pl.BlockSpec((1, D), lambda j: (0, 0)),
        ],
        out_specs=[
            pl.BlockSpec((PRE_BLK, 2 * D), lambda j: (j, 0)),
            pl.BlockSpec((8, 2 * D), lambda j: (0, 0)),
        ],
        out_shape=[
            jax.ShapeDtypeStruct((N_NODES, 2 * D), jnp.float32),
            jax.ShapeDtypeStruct((8, 2 * D), jnp.float32),
        ],
    )(feat, w2, r_table, wa_bot, bagg2)


def _sc_body(nodes_hbm, histvc_hbm, histr_hbm, p2_hbm, c8_hbm,
             x_out,
             nodes_v, idx_v, items_v, r_v, c_v, acc_v, selfbuf,
             rowa, rowb, semg, sem1, sema, semb):
    c = lax.axis_index("c")
    s = lax.axis_index("s")
    wid = s * 2 + c
    base = wid * BPT

    pltpu.sync_copy(nodes_hbm.at[pl.ds(base, BPT)], nodes_v)
    pltpu.sync_copy(c8_hbm, c_v)

    iota = lax.iota(jnp.int32, 16)

    def build(j, _):
        for k in range(CH // 16):
            g = j * CH + k * 16 + iota
            i = (g * 83887) >> 22
            l = g - i * L
            nd = plsc.load_gather(nodes_v, [i])
            idx_v[j, pl.ds(k * 16, 16)] = nd * L + l
        return 0

    lax.fori_loop(0, NCHUNK, build, 0, unroll=False)

    def zero(j, _):
        for k in range(D // 16):
            acc_v[j, pl.ds(k * 16, 16)] = jnp.zeros((16,), jnp.float32)
        return 0

    lax.fori_loop(0, BPT, zero, 0, unroll=False)

    def lvl1(j, _):
        ds = []
        for jj in range(10):
            row = j * 10 + jj
            ds.append(pltpu.async_copy(
                histvc_hbm.at[idx_v.at[row]], items_v.at[row], sem1))
            ds.append(pltpu.async_copy(
                histr_hbm.at[idx_v.at[row]], r_v.at[row], sem1))
        for dd in ds:
            dd.wait()
        return 0

    lax.fori_loop(0, NCHUNK // 10, lvl1, 0, unroll=False)

    pltpu.async_copy(p2_hbm.at[nodes_v], selfbuf, semg).wait()

    pltpu.async_copy(p2_hbm.at[items_v.at[0]], rowa, sema)
    pltpu.async_copy(p2_hbm.at[items_v.at[1]], rowb, semb)

    dnums = lax.GatherDimensionNumbers(
        offset_dims=(), collapsed_slice_dims=(0,), start_index_map=(0,))

    def process(chunk, buf):
        def group(g8, _):
            r16 = r_v[chunk, pl.ds(g8 * 16, 16)]
            gbase = chunk * CH + g8 * 16
            for i in range(16):
                node = ((gbase + i) * 83887) >> 22
                rsp = lax.gather(
                    r16, jnp.full((16, 1), i, jnp.int32), dnums, (1,),
                    mode=lax.GatherScatterMode.PROMISE_IN_BOUNDS)
                row = g8 * 16 + i
                for k in range(D // 16):
                    pv = buf[row, pl.ds(k * 16, 16)]
                    cv = plsc.load_gather(c_v, [rsp, iota + k * 16])
                    t = jnp.maximum(pv + cv, 0.0)
                    plsc.addupdate(acc_v.at[node, pl.ds(k * 16, 16)], t)
            return 0

        lax.fori_loop(0, CH // 16, group, 0, unroll=False)

    def main(j, _):
        c0 = 2 * j
        pltpu.make_async_copy(p2_hbm.at[items_v.at[c0]], rowa, sema).wait()
        process(c0, rowa)

        @pl.when(c0 + 2 < NCHUNK)
        def _fa():
            pltpu.async_copy(p2_hbm.at[items_v.at[c0 + 2]], rowa, sema)

        pltpu.make_async_copy(
            p2_hbm.at[items_v.at[c0 + 1]], rowb, semb).wait()
        process(c0 + 1, rowb)

        @pl.when(c0 + 3 < NCHUNK)
        def _fb():
            pltpu.async_copy(p2_hbm.at[items_v.at[c0 + 3]], rowb, semb)

        return 0

    lax.fori_loop(0, NCHUNK // 2, main, 0, unroll=False)

    def fin(i, _):
        for k in range(D // 16):
            selfbuf[i, pl.ds(k * 16, 16)] = (
                acc_v[i, pl.ds(k * 16, 16)] * (1.0 / L))
        return 0

    lax.fori_loop(0, BPT, fin, 0, unroll=False)
    pltpu.sync_copy(selfbuf, x_out.at[wid])


@jax.jit
def _sc_gather(nodes, histvc_flat, histr_flat, p2, c8):
    mesh = plsc.VectorSubcoreMesh(core_axis_name="c", subcore_axis_name="s")
    f = functools.partial(
        pl.kernel,
        compiler_params=pltpu.CompilerParams(
            use_tc_tiling_on_sc=False, needs_layout_passes=False,
            disable_bounds_checks=True),
        out_type=jax.ShapeDtypeStruct((NTILES, BPT, 2 * D), jnp.float32),
        mesh=mesh,
        scratch_types=[
            pltpu.VMEM((BPT,), jnp.int32),
            pltpu.VMEM((NCHUNK, CH), jnp.int32),
            pltpu.VMEM((NCHUNK, CH), jnp.int32),
            pltpu.VMEM((NCHUNK, CH), jnp.int32),
            pltpu.VMEM((8, 2 * D), jnp.float32),
            pltpu.VMEM((BPT, D), jnp.float32),
            pltpu.VMEM((BPT, 2 * D), jnp.float32),
            pltpu.VMEM((CH, 2 * D), jnp.float32),
            pltpu.VMEM((CH, 2 * D), jnp.float32),
            pltpu.SemaphoreType.DMA,
            pltpu.SemaphoreType.DMA,
            pltpu.SemaphoreType.DMA,
            pltpu.SemaphoreType.DMA,
        ],
    )(_sc_body)
    return f(nodes, histvc_flat, histr_flat, p2, c8)


def _fin_body(x_ref, w1bot_ref, b1_ref, out_ref):
    x = x_ref[...]
    out_ref[...] = jnp.maximum(
        x[:, D:] + jnp.dot(x[:, :D], w1bot_ref[...],
                           preferred_element_type=jnp.float32)
        + b1_ref[...], 0.0)


@jax.jit
def _tc_final(x2, w1_bot, b12):
    return pl.pallas_call(
        _fin_body,
        grid=(8,),
        in_specs=[
            pl.BlockSpec((B // 8, 2 * D), lambda j: (j, 0)),
            pl.BlockSpec((D, D), lambda j: (0, 0)),
            pl.BlockSpec((1, D), lambda j: (0, 0)),
        ],
        out_specs=pl.BlockSpec((B // 8, D), lambda j: (j, 0)),
        out_shape=jax.ShapeDtypeStruct((B, D), jnp.float32),
    )(x2, w1_bot, b12)


def kernel(nodes, hist_vc, hist_r, feat, r_table, W_agg, b_agg, W1, b1):
    w2 = jnp.concatenate([W_agg[:D], W1[:D]], axis=1)
    p2, c8 = _tc_prelude(feat, w2, r_table, W_agg[D:],
                         b_agg.reshape(1, D))
    x = _sc_gather(nodes, hist_vc.reshape(-1), hist_r.reshape(-1), p2, c8)
    return _tc_final(x.reshape(B, 2 * D), W1[D:], b1.reshape(1, D))


# speedup vs baseline: 1.2371x; 1.2371x over previous
import functools

import jax
import jax.numpy as jnp
from jax import lax
from jax.experimental import pallas as pl
from jax.experimental.pallas import tpu as pltpu
from jax.experimental.pallas import tpu_sc as plsc

N_NODES = 100000
D = 64
L = 50
B = 4096
NR = 5

NTILES = 32
BPT = B // NTILES
NCHUNK = L
CH = BPT
PRE_BLK = 1000


def _pre_body(feat_ref, w2_ref, rtab_ref, wabot_ref, bagg_ref,
              p2_ref, c8_ref):
    p2_ref[...] = jnp.dot(feat_ref[...], w2_ref[...],
                          preferred_element_type=jnp.float32)
    ct = jnp.dot(rtab_ref[...], wabot_ref[...],
                 preferred_element_type=jnp.float32) + bagg_ref[...]
    ct = jnp.concatenate([ct, jnp.zeros((8 - NR, D), jnp.float32)], axis=0)
    c8_ref[...] = jnp.concatenate([ct, jnp.zeros((8, D), jnp.float32)],
                                  axis=1)


@jax.jit
def _tc_prelude(feat, w2, r_table, wa_bot, bagg2):
    return pl.pallas_call(
        _pre_body,
        grid=(N_NODES // PRE_BLK,),
        in_specs=[
            pl.BlockSpec((PRE_BLK, D), lambda j: (j, 0)),
            pl.BlockSpec((D, 2 * D), lambda j: (0, 0)),
            pl.BlockSpec((NR, D), lambda j: (0, 0)),
            pl.BlockSpec((D, D), lambda j: (0, 0)),
            pl.BlockSpec((1, D), lambda j: (0, 0)),
        ],
        out_specs=[
            pl.BlockSpec((PRE_BLK, 2 * D), lambda j: (j, 0)),
            pl.BlockSpec((8, 2 * D), lambda j: (0, 0)),
        ],
        out_shape=[
            jax.ShapeDtypeStruct((N_NODES, 2 * D), jnp.float32),
            jax.ShapeDtypeStruct((8, 2 * D), jnp.float32),
        ],
    )(feat, w2, r_table, wa_bot, bagg2)


def _sc_body(nodes_hbm, histvc_hbm, histr_hbm, p2_hbm, c8_hbm,
             x_out,
             nodes_v, idx_v, items_v, r_v, c_v, acc_v, selfbuf,
             rowa, rowb, semg, sem1, sema, semb):
    c = lax.axis_index("c")
    s = lax.axis_index("s")
    wid = s * 2 + c
    base = wid * BPT

    pltpu.sync_copy(nodes_hbm.at[pl.ds(base, BPT)], nodes_v)
    pltpu.sync_copy(c8_hbm, c_v)

    iota = lax.iota(jnp.int32, 16)

    def build(j, _):
        for k in range(CH // 16):
            g = j * CH + k * 16 + iota
            i = (g * 83887) >> 22
            l = g - i * L
            nd = plsc.load_gather(nodes_v, [i])
            idx_v[j, pl.ds(k * 16, 16)] = nd * L + l
        return 0

    lax.fori_loop(0, NCHUNK, build, 0, unroll=False)

    def zero(j, _):
        for k in range(D // 16):
            acc_v[j, pl.ds(k * 16, 16)] = jnp.zeros((16,), jnp.float32)
        return 0

    lax.fori_loop(0, BPT, zero, 0, unroll=False)

    def lvl1(j, _):
        ds = []
        for jj in range(10):
            row = j * 10 + jj
            ds.append(pltpu.async_copy(
                histvc_hbm.at[idx_v.at[row]], items_v.at[row], sem1))
            ds.append(pltpu.async_copy(
                histr_hbm.at[idx_v.at[row]], r_v.at[row], sem1))
        for dd in ds:
            dd.wait()
        return 0

    lax.fori_loop(0, NCHUNK // 10, lvl1, 0, unroll=False)

    pltpu.async_copy(p2_hbm.at[nodes_v], selfbuf, semg).wait()

    pltpu.async_copy(p2_hbm.at[items_v.at[0]], rowa, sema)
    pltpu.async_copy(p2_hbm.at[items_v.at[1]], rowb, semb)

    dnums = lax.GatherDimensionNumbers(
        offset_dims=(), collapsed_slice_dims=(0,), start_index_map=(0,))

    def process(chunk, buf):
        def group(g8, _):
            r16 = r_v[chunk, pl.ds(g8 * 16, 16)]
            gbase = chunk * CH + g8 * 16
            nodes_sc = [((gbase + i) * 83887) >> 22 for i in range(16)]

            @plsc.parallel_loop(0, D // 16, step=1, unroll=4)
            def kloop(k):
                col = k * 16
                civ = iota + col
                for h in range(0, 16, 8):
                    rsps = [lax.gather(
                        r16, jnp.full((16, 1), i, jnp.int32), dnums, (1,),
                        mode=lax.GatherScatterMode.PROMISE_IN_BOUNDS)
                        for i in range(h, h + 8)]
                    pvs = [buf[g8 * 16 + h + j, pl.ds(col, 16)]
                           for j in range(8)]
                    cvs = [plsc.load_gather(c_v, [rsps[j], civ])
                           for j in range(8)]
                    for j in range(8):
                        t = jnp.maximum(pvs[j] + cvs[j], 0.0)
                        plsc.addupdate(
                            acc_v.at[nodes_sc[h + j], pl.ds(col, 16)], t)

            return 0

        lax.fori_loop(0, CH // 16, group, 0, unroll=False)

    def main(j, _):
        c0 = 2 * j
        pltpu.make_async_copy(p2_hbm.at[items_v.at[c0]], rowa, sema).wait()
        process(c0, rowa)

        @pl.when(c0 + 2 < NCHUNK)
        def _fa():
            pltpu.async_copy(p2_hbm.at[items_v.at[c0 + 2]], rowa, sema)

        pltpu.make_async_copy(
            p2_hbm.at[items_v.at[c0 + 1]], rowb, semb).wait()
        process(c0 + 1, rowb)

        @pl.when(c0 + 3 < NCHUNK)
        def _fb():
            pltpu.async_copy(p2_hbm.at[items_v.at[c0 + 3]], rowb, semb)

        return 0

    lax.fori_loop(0, NCHUNK // 2, main, 0, unroll=False)

    def fin(i, _):
        for k in range(D // 16):
            selfbuf[i, pl.ds(k * 16, 16)] = (
                acc_v[i, pl.ds(k * 16, 16)] * (1.0 / L))
        return 0

    lax.fori_loop(0, BPT, fin, 0, unroll=False)
    pltpu.sync_copy(selfbuf, x_out.at[wid])


@jax.jit
def _sc_gather(nodes, histvc_flat, histr_flat, p2, c8):
    mesh = plsc.VectorSubcoreMesh(core_axis_name="c", subcore_axis_name="s")
    f = functools.partial(
        pl.kernel,
        compiler_params=pltpu.CompilerParams(
            use_tc_tiling_on_sc=False, needs_layout_passes=False,
            disable_bounds_checks=True),
        out_type=jax.ShapeDtypeStruct((NTILES, BPT, 2 * D), jnp.float32),
        mesh=mesh,
        scratch_types=[
            pltpu.VMEM((BPT,), jnp.int32),
            pltpu.VMEM((NCHUNK, CH), jnp.int32),
            pltpu.VMEM((NCHUNK, CH), jnp.int32),
            pltpu.VMEM((NCHUNK, CH), jnp.int32),
            pltpu.VMEM((8, 2 * D), jnp.float32),
            pltpu.VMEM((BPT, D), jnp.float32),
            pltpu.VMEM((BPT, 2 * D), jnp.float32),
            pltpu.VMEM((CH, 2 * D), jnp.float32),
            pltpu.VMEM((CH, 2 * D), jnp.float32),
            pltpu.SemaphoreType.DMA,
            pltpu.SemaphoreType.DMA,
            pltpu.SemaphoreType.DMA,
            pltpu.SemaphoreType.DMA,
        ],
    )(_sc_body)
    return f(nodes, histvc_flat, histr_flat, p2, c8)


def _fin_body(x_ref, w1bot_ref, b1_ref, out_ref):
    x = x_ref[...]
    out_ref[...] = jnp.maximum(
        x[:, D:] + jnp.dot(x[:, :D], w1bot_ref[...],
                           preferred_element_type=jnp.float32)
        + b1_ref[...], 0.0)


@jax.jit
def _tc_final(x2, w1_bot, b12):
    return pl.pallas_call(
        _fin_body,
        grid=(8,),
        in_specs=[
            pl.BlockSpec((B // 8, 2 * D), lambda j: (j, 0)),
            pl.BlockSpec((D, D), lambda j: (0, 0)),
            pl.BlockSpec((1, D), lambda j: (0, 0)),
        ],
        out_specs=pl.BlockSpec((B // 8, D), lambda j: (j, 0)),
        out_shape=jax.ShapeDtypeStruct((B, D), jnp.float32),
    )(x2, w1_bot, b12)


def kernel(nodes, hist_vc, hist_r, feat, r_table, W_agg, b_agg, W1, b1):
    w2 = jnp.concatenate([W_agg[:D], W1[:D]], axis=1)
    p2, c8 = _tc_prelude(feat, w2, r_table, W_agg[D:],
                         b_agg.reshape(1, D))
    x = _sc_gather(nodes, hist_vc.reshape(-1), hist_r.reshape(-1), p2, c8)
    return _tc_final(x.reshape(B, 2 * D), W1[D:], b1.reshape(1, D))


# speedup vs baseline: 1.5592x; 1.2603x over previous
import functools

import jax
import jax.numpy as jnp
from jax import lax
from jax.experimental import pallas as pl
from jax.experimental.pallas import tpu as pltpu
from jax.experimental.pallas import tpu_sc as plsc

N_NODES = 100000
D = 64
L = 50
B = 4096
NR = 5

NTILES = 32
BPT = B // NTILES
NCHUNK = L
CH = BPT
PRE_BLK = 2000


def _pre_body(feat_ref, vc_ref, hr_ref, w2_ref, rtab_ref, wabot_ref,
              bagg_ref, p2_ref, h2_ref, c8_ref):
    p2_ref[...] = jnp.dot(feat_ref[...], w2_ref[...],
                          preferred_element_type=jnp.float32)
    h2_ref[...] = jnp.concatenate(
        [vc_ref[...], hr_ref[...],
         jnp.zeros((PRE_BLK, 2 * D - 2 * L), jnp.int32)], axis=1)
    ct = jnp.dot(rtab_ref[...], wabot_ref[...],
                 preferred_element_type=jnp.float32) + bagg_ref[...]
    ct = jnp.concatenate([ct, jnp.zeros((8 - NR, D), jnp.float32)], axis=0)
    c8_ref[...] = jnp.concatenate([ct, jnp.zeros((8, D), jnp.float32)],
                                  axis=1)


@jax.jit
def _tc_prelude(feat, hist_vc, hist_r, w2, r_table, wa_bot, bagg2):
    return pl.pallas_call(
        _pre_body,
        grid=(N_NODES // PRE_BLK,),
        in_specs=[
            pl.BlockSpec((PRE_BLK, D), lambda j: (j, 0)),
            pl.BlockSpec((PRE_BLK, L), lambda j: (j, 0)),
            pl.BlockSpec((PRE_BLK, L), lambda j: (j, 0)),
            pl.BlockSpec((D, 2 * D), lambda j: (0, 0)),
            pl.BlockSpec((NR, D), lambda j: (0, 0)),
            pl.BlockSpec((D, D), lambda j: (0, 0)),
            pl.BlockSpec((1, D), lambda j: (0, 0)),
        ],
        out_specs=[
            pl.BlockSpec((PRE_BLK, 2 * D), lambda j: (j, 0)),
            pl.BlockSpec((PRE_BLK, 2 * D), lambda j: (j, 0)),
            pl.BlockSpec((8, 2 * D), lambda j: (0, 0)),
        ],
        out_shape=[
            jax.ShapeDtypeStruct((N_NODES, 2 * D), jnp.float32),
            jax.ShapeDtypeStruct((N_NODES, 2 * D), jnp.int32),
            jax.ShapeDtypeStruct((8, 2 * D), jnp.float32),
        ],
    )(feat, hist_vc, hist_r, w2, r_table, wa_bot, bagg2)


def _sc_body(nodes_hbm, h2_hbm, p2_hbm, c8_hbm,
             x_out,
             nodes_v, hrows, items_v, c_v, acc_v, selfbuf,
             rowa, rowb, semg, sem1, sema, semb):
    c = lax.axis_index("c")
    s = lax.axis_index("s")
    wid = s * 2 + c
    base = wid * BPT

    pltpu.sync_copy(nodes_hbm.at[pl.ds(base, BPT)], nodes_v)
    pltpu.sync_copy(c8_hbm, c_v)

    iota = lax.iota(jnp.int32, 16)

    d1 = pltpu.async_copy(h2_hbm.at[nodes_v], hrows, sem1)

    def zero(j, _):
        for k in range(D // 16):
            acc_v[j, pl.ds(k * 16, 16)] = jnp.zeros((16,), jnp.float32)
        return 0

    lax.fori_loop(0, BPT, zero, 0, unroll=False)
    d1.wait()

    def build(j, _):
        for k in range(CH // 16):
            g = j * CH + k * 16 + iota
            nv = (g * 83887) >> 22
            lv = g - nv * L
            items_v[j, pl.ds(k * 16, 16)] = plsc.load_gather(
                hrows, [nv, lv])
        return 0

    lax.fori_loop(0, NCHUNK, build, 0, unroll=False)

    pltpu.async_copy(p2_hbm.at[nodes_v], selfbuf, semg).wait()

    pltpu.async_copy(p2_hbm.at[items_v.at[0]], rowa, sema)
    pltpu.async_copy(p2_hbm.at[items_v.at[1]], rowb, semb)

    dnums = lax.GatherDimensionNumbers(
        offset_dims=(), collapsed_slice_dims=(0,), start_index_map=(0,))

    def process(chunk, buf):
        def group(g8, _):
            gv = chunk * CH + g8 * 16 + iota
            nv = (gv * 83887) >> 22
            r16 = plsc.load_gather(hrows, [nv, gv - nv * L + L])
            gbase = chunk * CH + g8 * 16
            nodes_sc = [((gbase + i) * 83887) >> 22 for i in range(16)]

            @plsc.parallel_loop(0, D // 16, step=1, unroll=4)
            def kloop(k):
                col = k * 16
                civ = iota + col
                for h in range(0, 16, 8):
                    rsps = [lax.gather(
                        r16, jnp.full((16, 1), i, jnp.int32), dnums, (1,),
                        mode=lax.GatherScatterMode.PROMISE_IN_BOUNDS)
                        for i in range(h, h + 8)]
                    pvs = [buf[g8 * 16 + h + j, pl.ds(col, 16)]
                           for j in range(8)]
                    cvs = [plsc.load_gather(c_v, [rsps[j], civ])
                           for j in range(8)]
                    for j in range(8):
                        t = jnp.maximum(pvs[j] + cvs[j], 0.0)
                        plsc.addupdate(
                            acc_v.at[nodes_sc[h + j], pl.ds(col, 16)], t)

            return 0

        lax.fori_loop(0, CH // 16, group, 0, unroll=False)

    def main(j, _):
        c0 = 2 * j
        pltpu.make_async_copy(p2_hbm.at[items_v.at[c0]], rowa, sema).wait()
        process(c0, rowa)

        @pl.when(c0 + 2 < NCHUNK)
        def _fa():
            pltpu.async_copy(p2_hbm.at[items_v.at[c0 + 2]], rowa, sema)

        pltpu.make_async_copy(
            p2_hbm.at[items_v.at[c0 + 1]], rowb, semb).wait()
        process(c0 + 1, rowb)

        @pl.when(c0 + 3 < NCHUNK)
        def _fb():
            pltpu.async_copy(p2_hbm.at[items_v.at[c0 + 3]], rowb, semb)

        return 0

    lax.fori_loop(0, NCHUNK // 2, main, 0, unroll=False)

    def fin(i, _):
        for k in range(D // 16):
            selfbuf[i, pl.ds(k * 16, 16)] = (
                acc_v[i, pl.ds(k * 16, 16)] * (1.0 / L))
        return 0

    lax.fori_loop(0, BPT, fin, 0, unroll=False)
    pltpu.sync_copy(selfbuf, x_out.at[wid])


@jax.jit
def _sc_gather(nodes, h2, p2, c8):
    mesh = plsc.VectorSubcoreMesh(core_axis_name="c", subcore_axis_name="s")
    f = functools.partial(
        pl.kernel,
        compiler_params=pltpu.CompilerParams(
            use_tc_tiling_on_sc=False, needs_layout_passes=False,
            disable_bounds_checks=True),
        out_type=jax.ShapeDtypeStruct((NTILES, BPT, 2 * D), jnp.float32),
        mesh=mesh,
        scratch_types=[
            pltpu.VMEM((BPT,), jnp.int32),
            pltpu.VMEM((BPT, 2 * D), jnp.int32),
            pltpu.VMEM((NCHUNK, CH), jnp.int32),
            pltpu.VMEM((8, 2 * D), jnp.float32),
            pltpu.VMEM((BPT, D), jnp.float32),
            pltpu.VMEM((BPT, 2 * D), jnp.float32),
            pltpu.VMEM((CH, 2 * D), jnp.float32),
            pltpu.VMEM((CH, 2 * D), jnp.float32),
            pltpu.SemaphoreType.DMA,
            pltpu.SemaphoreType.DMA,
            pltpu.SemaphoreType.DMA,
            pltpu.SemaphoreType.DMA,
        ],
    )(_sc_body)
    return f(nodes, h2, p2, c8)


def _fin_body(x_ref, w1bot_ref, b1_ref, out_ref):
    x = x_ref[...]
    out_ref[...] = jnp.maximum(
        x[:, D:] + jnp.dot(x[:, :D], w1bot_ref[...],
                           preferred_element_type=jnp.float32)
        + b1_ref[...], 0.0)


@jax.jit
def _tc_final(x2, w1_bot, b12):
    return pl.pallas_call(
        _fin_body,
        grid=(8,),
        in_specs=[
            pl.BlockSpec((B // 8, 2 * D), lambda j: (j, 0)),
            pl.BlockSpec((D, D), lambda j: (0, 0)),
            pl.BlockSpec((1, D), lambda j: (0, 0)),
        ],
        out_specs=pl.BlockSpec((B // 8, D), lambda j: (j, 0)),
        out_shape=jax.ShapeDtypeStruct((B, D), jnp.float32),
    )(x2, w1_bot, b12)


def kernel(nodes, hist_vc, hist_r, feat, r_table, W_agg, b_agg, W1, b1):
    w2 = jnp.concatenate([W_agg[:D], W1[:D]], axis=1)
    p2, h2, c8 = _tc_prelude(feat, hist_vc, hist_r, w2, r_table, W_agg[D:],
                             b_agg.reshape(1, D))
    x = _sc_gather(nodes, h2, p2, c8)
    return _tc_final(x.reshape(B, 2 * D), W1[D:], b1.reshape(1, D))


# speedup vs baseline: 2.4431x; 1.5669x over previous
import functools

import jax
import jax.numpy as jnp
from jax import lax
from jax.experimental import pallas as pl
from jax.experimental.pallas import tpu as pltpu
from jax.experimental.pallas import tpu_sc as plsc

N_NODES = 100000
D = 64
L = 50
B = 4096
NR = 5

NTILES = 32
BPT = B // NTILES
NCHUNK = L
CH = BPT
PRE_BLK = 2048


def _pre_body(feat_ref, vc_ref, hr_ref, w2_ref, rtab_ref, wabot_ref,
              bagg_ref, p2_ref, h2_ref, c8_ref):
    p2_ref[...] = lax.dot_general(
        feat_ref[...], w2_ref[...], (((0,), (0,)), ((), ())),
        preferred_element_type=jnp.float32)
    h2_ref[...] = jnp.concatenate(
        [vc_ref[...].T, hr_ref[...].T,
         jnp.zeros((PRE_BLK, 2 * D - 2 * L), jnp.int32)], axis=1)
    ct = jnp.dot(rtab_ref[...], wabot_ref[...],
                 preferred_element_type=jnp.float32) + bagg_ref[...]
    ct = jnp.concatenate([ct, jnp.zeros((8 - NR, D), jnp.float32)], axis=0)
    c8_ref[...] = jnp.concatenate([ct, jnp.zeros((8, D), jnp.float32)],
                                  axis=1)


@jax.jit
def _tc_prelude(feat_t, vc_t, hr_t, w2, r_table, wa_bot, bagg2):
    return pl.pallas_call(
        _pre_body,
        grid=(pl.cdiv(N_NODES, PRE_BLK),),
        in_specs=[
            pl.BlockSpec((D, PRE_BLK), lambda j: (0, j)),
            pl.BlockSpec((L, PRE_BLK), lambda j: (0, j)),
            pl.BlockSpec((L, PRE_BLK), lambda j: (0, j)),
            pl.BlockSpec((D, 2 * D), lambda j: (0, 0)),
            pl.BlockSpec((NR, D), lambda j: (0, 0)),
            pl.BlockSpec((D, D), lambda j: (0, 0)),
            pl.BlockSpec((1, D), lambda j: (0, 0)),
        ],
        out_specs=[
            pl.BlockSpec((PRE_BLK, 2 * D), lambda j: (j, 0)),
            pl.BlockSpec((PRE_BLK, 2 * D), lambda j: (j, 0)),
            pl.BlockSpec((8, 2 * D), lambda j: (0, 0)),
        ],
        out_shape=[
            jax.ShapeDtypeStruct((N_NODES, 2 * D), jnp.float32),
            jax.ShapeDtypeStruct((N_NODES, 2 * D), jnp.int32),
            jax.ShapeDtypeStruct((8, 2 * D), jnp.float32),
        ],
    )(feat_t, vc_t, hr_t, w2, r_table, wa_bot, bagg2)


def _sc_body(nodes_hbm, h2_hbm, p2_hbm, c8_hbm,
             x_out,
             nodes_v, hrows, items_v, c_v, acc_v, selfbuf,
             rowa, rowb, semg, sem1, sema, semb):
    c = lax.axis_index("c")
    s = lax.axis_index("s")
    wid = s * 2 + c
    base = wid * BPT

    pltpu.sync_copy(nodes_hbm.at[pl.ds(base, BPT)], nodes_v)
    pltpu.sync_copy(c8_hbm, c_v)

    iota = lax.iota(jnp.int32, 16)

    d1 = pltpu.async_copy(h2_hbm.at[nodes_v], hrows, sem1)

    def zero(j, _):
        for k in range(D // 16):
            acc_v[j, pl.ds(k * 16, 16)] = jnp.zeros((16,), jnp.float32)
        return 0

    lax.fori_loop(0, BPT, zero, 0, unroll=False)
    d1.wait()

    def build(j, _):
        for k in range(CH // 16):
            g = j * CH + k * 16 + iota
            nv = (g * 83887) >> 22
            lv = g - nv * L
            items_v[j, pl.ds(k * 16, 16)] = plsc.load_gather(
                hrows, [nv, lv])
        return 0

    lax.fori_loop(0, NCHUNK, build, 0, unroll=False)

    pltpu.async_copy(p2_hbm.at[nodes_v], selfbuf, semg).wait()

    pltpu.async_copy(p2_hbm.at[items_v.at[0]], rowa, sema)
    pltpu.async_copy(p2_hbm.at[items_v.at[1]], rowb, semb)

    dnums = lax.GatherDimensionNumbers(
        offset_dims=(), collapsed_slice_dims=(0,), start_index_map=(0,))

    def process(chunk, buf):
        def group(g8, _):
            gv = chunk * CH + g8 * 16 + iota
            nv = (gv * 83887) >> 22
            r16 = plsc.load_gather(hrows, [nv, gv - nv * L + L])
            gbase = chunk * CH + g8 * 16
            nodes_sc = [((gbase + i) * 83887) >> 22 for i in range(16)]

            @plsc.parallel_loop(0, D // 16, step=1, unroll=4)
            def kloop(k):
                col = k * 16
                civ = iota + col
                for h in range(0, 16, 8):
                    rsps = [lax.gather(
                        r16, jnp.full((16, 1), i, jnp.int32), dnums, (1,),
                        mode=lax.GatherScatterMode.PROMISE_IN_BOUNDS)
                        for i in range(h, h + 8)]
                    pvs = [buf[g8 * 16 + h + j, pl.ds(col, 16)]
                           for j in range(8)]
                    cvs = [plsc.load_gather(c_v, [rsps[j], civ])
                           for j in range(8)]
                    for j in range(8):
                        t = jnp.maximum(pvs[j] + cvs[j], 0.0)
                        plsc.addupdate(
                            acc_v.at[nodes_sc[h + j], pl.ds(col, 16)], t)

            return 0

        lax.fori_loop(0, CH // 16, group, 0, unroll=False)

    def main(j, _):
        c0 = 2 * j
        pltpu.make_async_copy(p2_hbm.at[items_v.at[c0]], rowa, sema).wait()
        process(c0, rowa)

        @pl.when(c0 + 2 < NCHUNK)
        def _fa():
            pltpu.async_copy(p2_hbm.at[items_v.at[c0 + 2]], rowa, sema)

        pltpu.make_async_copy(
            p2_hbm.at[items_v.at[c0 + 1]], rowb, semb).wait()
        process(c0 + 1, rowb)

        @pl.when(c0 + 3 < NCHUNK)
        def _fb():
            pltpu.async_copy(p2_hbm.at[items_v.at[c0 + 3]], rowb, semb)

        return 0

    lax.fori_loop(0, NCHUNK // 2, main, 0, unroll=False)

    def fin(i, _):
        for k in range(D // 16):
            selfbuf[i, pl.ds(k * 16, 16)] = (
                acc_v[i, pl.ds(k * 16, 16)] * (1.0 / L))
        return 0

    lax.fori_loop(0, BPT, fin, 0, unroll=False)
    pltpu.sync_copy(selfbuf, x_out.at[wid])


@jax.jit
def _sc_gather(nodes, h2, p2, c8):
    mesh = plsc.VectorSubcoreMesh(core_axis_name="c", subcore_axis_name="s")
    f = functools.partial(
        pl.kernel,
        compiler_params=pltpu.CompilerParams(
            use_tc_tiling_on_sc=False, needs_layout_passes=False,
            disable_bounds_checks=True),
        out_type=jax.ShapeDtypeStruct((NTILES, BPT, 2 * D), jnp.float32),
        mesh=mesh,
        scratch_types=[
            pltpu.VMEM((BPT,), jnp.int32),
            pltpu.VMEM((BPT, 2 * D), jnp.int32),
            pltpu.VMEM((NCHUNK, CH), jnp.int32),
            pltpu.VMEM((8, 2 * D), jnp.float32),
            pltpu.VMEM((BPT, D), jnp.float32),
            pltpu.VMEM((BPT, 2 * D), jnp.float32),
            pltpu.VMEM((CH, 2 * D), jnp.float32),
            pltpu.VMEM((CH, 2 * D), jnp.float32),
            pltpu.SemaphoreType.DMA,
            pltpu.SemaphoreType.DMA,
            pltpu.SemaphoreType.DMA,
            pltpu.SemaphoreType.DMA,
        ],
    )(_sc_body)
    return f(nodes, h2, p2, c8)


def _fin_body(x_ref, w1bot_ref, b1_ref, out_ref):
    x = x_ref[...]
    out_ref[...] = jnp.maximum(
        x[:, D:] + jnp.dot(x[:, :D], w1bot_ref[...],
                           preferred_element_type=jnp.float32)
        + b1_ref[...], 0.0)


@jax.jit
def _tc_final(x2, w1_bot, b12):
    return pl.pallas_call(
        _fin_body,
        grid=(8,),
        in_specs=[
            pl.BlockSpec((B // 8, 2 * D), lambda j: (j, 0)),
            pl.BlockSpec((D, D), lambda j: (0, 0)),
            pl.BlockSpec((1, D), lambda j: (0, 0)),
        ],
        out_specs=pl.BlockSpec((B // 8, D), lambda j: (j, 0)),
        out_shape=jax.ShapeDtypeStruct((B, D), jnp.float32),
    )(x2, w1_bot, b12)


def kernel(nodes, hist_vc, hist_r, feat, r_table, W_agg, b_agg, W1, b1):
    w2 = jnp.concatenate([W_agg[:D], W1[:D]], axis=1)
    p2, h2, c8 = _tc_prelude(feat.T, hist_vc.T, hist_r.T, w2, r_table,
                             W_agg[D:], b_agg.reshape(1, D))
    x = _sc_gather(nodes, h2, p2, c8)
    return _tc_final(x.reshape(B, 2 * D), W1[D:], b1.reshape(1, D))
